# Initial kernel scaffold; baseline (speedup 1.0000x reference)
import jax
import jax.numpy as jnp
from jax.experimental import pallas as pl


def kernel(X_in, weights, input_node_order):
    raise NotImplementedError("write your pallas kernel here")


# speedup vs baseline: 1.0204x; 1.0204x over previous
import jax
import jax.numpy as jnp
from jax import lax
from jax.experimental import pallas as pl
from jax.experimental.pallas import tpu as pltpu
from jax.experimental.pallas import tpu_sc as plsc

_BATCH = 32768
_NIN = 64
_NOUT = 256
_NC = 2
_NS = 16
_NW = _NC * _NS
_ROWS_PER_W = _BATCH // _NW
_CHUNK = 128
_NCHUNKS = _ROWS_PER_W // _CHUNK
_L = 16
_G = _NIN // _L


def _sc_body(x_hbm, w_hbm, idx_hbm, out_hbm, x_v, out_v, w_v, idx_v):
    wid = lax.axis_index("s") * _NC + lax.axis_index("c")
    base_row = wid * _ROWS_PER_W

    pltpu.sync_copy(w_hbm, w_v)
    pltpu.sync_copy(idx_hbm, idx_v)

    zero = jnp.zeros((_L,), jnp.float32)

    def zero_body(i, carry):
        out_v[pl.ds(i * _L, _L)] = zero
        return carry

    lax.fori_loop(0, _CHUNK * _NOUT // _L, zero_body, 0)

    w_regs = tuple(w_v[pl.ds(g * _L, _L)] for g in range(_G))
    idx_regs = tuple(idx_v[pl.ds(g * _L, _L)] for g in range(_G))
    row_step = jnp.full((_L,), _NOUT, jnp.int32)

    def chunk_body(ci, carry):
        w_c, idx_c = carry
        row0 = base_row + ci * _CHUNK
        pltpu.sync_copy(x_hbm.at[pl.ds(row0 * _NIN, _CHUNK * _NIN)], x_v)

        def row_body(r, idx_roll):
            for g in range(_G):
                vals = x_v[pl.ds(r * _NIN + g * _L, _L)] * w_c[g]
                plsc.store_scatter(out_v, [idx_roll[g]], vals)
            return tuple(ix + row_step for ix in idx_roll)

        lax.fori_loop(0, _CHUNK, row_body, idx_c)
        pltpu.sync_copy(out_v, out_hbm.at[pl.ds(row0 * _NOUT, _CHUNK * _NOUT)])
        return carry

    lax.fori_loop(0, _NCHUNKS, chunk_body, (w_regs, idx_regs))


def kernel(X_in, weights, input_node_order):
    mesh = plsc.VectorSubcoreMesh(
        core_axis_name="c", subcore_axis_name="s",
        num_cores=_NC, num_subcores=_NS,
    )
    f = pl.kernel(
        _sc_body,
        out_type=jax.ShapeDtypeStruct((_BATCH * _NOUT,), jnp.float32),
        mesh=mesh,
        compiler_params=pltpu.CompilerParams(needs_layout_passes=False),
        scratch_types=[
            pltpu.VMEM((_CHUNK * _NIN,), jnp.float32),
            pltpu.VMEM((_CHUNK * _NOUT,), jnp.float32),
            pltpu.VMEM((_NIN,), jnp.float32),
            pltpu.VMEM((_NIN,), jnp.int32),
        ],
    )
    flat = f(X_in.reshape(_BATCH * _NIN), weights, input_node_order)
    return flat.reshape(_BATCH, _NOUT)


# speedup vs baseline: 1.6197x; 1.5873x over previous
import jax
import jax.numpy as jnp
from jax import lax
from jax.experimental import pallas as pl
from jax.experimental.pallas import tpu as pltpu
from jax.experimental.pallas import tpu_sc as plsc

_BATCH = 32768
_NIN = 64
_NOUT = 256
_NC = 2
_NS = 16
_NW = _NC * _NS
_ROWS_PER_W = _BATCH // _NW
_CHUNK = 128
_NCHUNKS = _ROWS_PER_W // _CHUNK
_L = 16
_G = _NIN // _L


def _sc_body(x_hbm, w_hbm, idx_hbm, out_hbm, x_v, out_v, w_v, idx_v):
    wid = lax.axis_index("s") * _NC + lax.axis_index("c")
    base_row = wid * _ROWS_PER_W

    pltpu.sync_copy(w_hbm, w_v)
    pltpu.sync_copy(idx_hbm, idx_v)

    zero = jnp.zeros((_L,), jnp.float32)

    def zero_body(i, carry):
        r = i // (_NOUT // _L)
        k = i % (_NOUT // _L)
        out_v[r, pl.ds(k * _L, _L)] = zero
        return carry

    lax.fori_loop(0, _CHUNK * _NOUT // _L, zero_body, 0)

    w_regs = tuple(w_v[pl.ds(g * _L, _L)] for g in range(_G))
    idx_regs = tuple(idx_v[pl.ds(g * _L, _L)] for g in range(_G))

    def chunk_body(ci, carry):
        w_c, idx_c = carry
        row0 = base_row + ci * _CHUNK
        pltpu.sync_copy(x_hbm.at[pl.ds(row0, _CHUNK)], x_v)

        def row_body(r, carry2):
            w_r, idx_r = carry2
            rsplat = jnp.full((_L,), r, jnp.int32)
            for g in range(_G):
                vals = x_v[r, pl.ds(g * _L, _L)] * w_r[g]
                plsc.store_scatter(out_v, [rsplat, idx_r[g]], vals)
            return carry2

        lax.fori_loop(0, _CHUNK, row_body, (w_c, idx_c))
        pltpu.sync_copy(out_v, out_hbm.at[pl.ds(row0, _CHUNK)])
        return carry

    lax.fori_loop(0, _NCHUNKS, chunk_body, (w_regs, idx_regs))


def kernel(X_in, weights, input_node_order):
    mesh = plsc.VectorSubcoreMesh(
        core_axis_name="c", subcore_axis_name="s",
        num_cores=_NC, num_subcores=_NS,
    )
    f = pl.kernel(
        _sc_body,
        out_type=jax.ShapeDtypeStruct((_BATCH, _NOUT), jnp.float32),
        mesh=mesh,
        compiler_params=pltpu.CompilerParams(needs_layout_passes=False),
        scratch_types=[
            pltpu.VMEM((_CHUNK, _NIN), jnp.float32),
            pltpu.VMEM((_CHUNK, _NOUT), jnp.float32),
            pltpu.VMEM((_NIN,), jnp.float32),
            pltpu.VMEM((_NIN,), jnp.int32),
        ],
    )
    return f(X_in, weights, input_node_order)


# speedup vs baseline: 2.5378x; 1.5669x over previous
import jax
import jax.numpy as jnp
from jax import lax
from jax.experimental import pallas as pl
from jax.experimental.pallas import tpu as pltpu
from jax.experimental.pallas import tpu_sc as plsc

_BATCH = 32768
_NIN = 64
_NOUT = 256
_NC = 2
_NS = 16
_NW = _NC * _NS
_ROWS_PER_W = _BATCH // _NW
_CHUNK = 128
_NCHUNKS = _ROWS_PER_W // _CHUNK
_L = 16
_G = _NIN // _L
_U = 4


def _sc_body(x_hbm, w_hbm, idx_hbm, out_hbm,
             x_v0, x_v1, out_v0, out_v1, w_v, idx_v,
             sem_x0, sem_x1, sem_o0, sem_o1):
    wid = lax.axis_index("s") * _NC + lax.axis_index("c")
    base_row = wid * _ROWS_PER_W

    x_bufs = (x_v0, x_v1)
    out_bufs = (out_v0, out_v1)
    x_sems = (sem_x0, sem_x1)
    o_sems = (sem_o0, sem_o1)

    x_dma0 = pltpu.async_copy(x_hbm.at[pl.ds(base_row, _CHUNK)], x_v0, sem_x0)

    pltpu.sync_copy(w_hbm, w_v)
    pltpu.sync_copy(idx_hbm, idx_v)

    zero = jnp.zeros((_L,), jnp.float32)

    def zero_body(i, carry):
        r = i // (_NOUT // _L)
        k = (i % (_NOUT // _L)) * _L
        for b in range(2):
            out_bufs[b][r, pl.ds(k, _L)] = zero
            out_bufs[b][r + 1, pl.ds(k, _L)] = zero
        return carry

    lax.fori_loop(0, _CHUNK // 2 * (_NOUT // _L), zero_body, 0,
                  unroll=4)

    w_regs = tuple(w_v[pl.ds(g * _L, _L)] for g in range(_G))
    idx_regs = tuple(idx_v[pl.ds(g * _L, _L)] for g in range(_G))

    x_dmas = [x_dma0, None]
    o_dmas = [None, None]
    for ci in range(_NCHUNKS):
        b = ci % 2
        row0 = base_row + ci * _CHUNK
        if ci + 1 < _NCHUNKS:
            nb = (ci + 1) % 2
            x_dmas[nb] = pltpu.async_copy(
                x_hbm.at[pl.ds(row0 + _CHUNK, _CHUNK)], x_bufs[nb], x_sems[nb])
        x_dmas[b].wait()
        if o_dmas[b] is not None:
            o_dmas[b].wait()

        x_v = x_bufs[b]
        out_v = out_bufs[b]

        def row_body(i, carry):
            w_r, idx_r = carry
            for u in range(_U):
                r = i * _U + u
                vals = tuple(x_v[r, pl.ds(g * _L, _L)] * w_r[g]
                             for g in range(_G))
                rsplat = jnp.full((_L,), r, jnp.int32)
                for g in range(_G):
                    plsc.store_scatter(out_v, [rsplat, idx_r[g]], vals[g])
            return carry

        lax.fori_loop(0, _CHUNK // _U, row_body, (w_regs, idx_regs))

        o_dmas[b] = pltpu.async_copy(
            out_v, out_hbm.at[pl.ds(row0, _CHUNK)], o_sems[b])

    for d in o_dmas:
        if d is not None:
            d.wait()


def kernel(X_in, weights, input_node_order):
    mesh = plsc.VectorSubcoreMesh(
        core_axis_name="c", subcore_axis_name="s",
        num_cores=_NC, num_subcores=_NS,
    )
    f = pl.kernel(
        _sc_body,
        out_type=jax.ShapeDtypeStruct((_BATCH, _NOUT), jnp.float32),
        mesh=mesh,
        compiler_params=pltpu.CompilerParams(needs_layout_passes=False),
        scratch_types=[
            pltpu.VMEM((_CHUNK, _NIN), jnp.float32),
            pltpu.VMEM((_CHUNK, _NIN), jnp.float32),
            pltpu.VMEM((_CHUNK, _NOUT), jnp.float32),
            pltpu.VMEM((_CHUNK, _NOUT), jnp.float32),
            pltpu.VMEM((_NIN,), jnp.float32),
            pltpu.VMEM((_NIN,), jnp.int32),
            pltpu.SemaphoreType.DMA,
            pltpu.SemaphoreType.DMA,
            pltpu.SemaphoreType.DMA,
            pltpu.SemaphoreType.DMA,
        ],
    )
    return f(X_in, weights, input_node_order)


# speedup vs baseline: 2.5380x; 1.0001x over previous
import jax
import jax.numpy as jnp
from jax import lax
from jax.experimental import pallas as pl
from jax.experimental.pallas import tpu as pltpu
from jax.experimental.pallas import tpu_sc as plsc

_BATCH = 32768
_NIN = 64
_NOUT = 256
_NC = 2
_NS = 16
_NW = _NC * _NS
_ROWS_PER_W = _BATCH // _NW
_CHUNK = 128
_NCHUNKS = _ROWS_PER_W // _CHUNK
_L = 16
_G = _NIN // _L
_U = 4


def _sc_body(x_hbm, w_hbm, idx_hbm, out_hbm,
             x_v0, x_v1, out_v0, out_v1, w_v, idx_v,
             sem_x0, sem_x1, sem_o0, sem_o1):
    wid = lax.axis_index("s") * _NC + lax.axis_index("c")
    base_row = wid * _ROWS_PER_W

    x_bufs = (x_v0, x_v1)
    out_bufs = (out_v0, out_v1)
    x_sems = (sem_x0, sem_x1)
    o_sems = (sem_o0, sem_o1)

    x_dma0 = pltpu.async_copy(x_hbm.at[pl.ds(base_row, _CHUNK)], x_v0, sem_x0)

    pltpu.sync_copy(w_hbm, w_v)
    pltpu.sync_copy(idx_hbm, idx_v)

    zero = jnp.zeros((_L,), jnp.float32)

    def zero_body(i, carry):
        r = i // (_NOUT // _L)
        k = (i % (_NOUT // _L)) * _L
        for b in range(2):
            out_bufs[b][r, pl.ds(k, _L)] = zero
            out_bufs[b][r + 1, pl.ds(k, _L)] = zero
        return carry

    lax.fori_loop(0, _CHUNK // 2 * (_NOUT // _L), zero_body, 0,
                  unroll=4)

    w_regs = tuple(w_v[pl.ds(g * _L, _L)] for g in range(_G))
    idx_regs = tuple(idx_v[pl.ds(g * _L, _L)] for g in range(_G))

    x_dmas = [x_dma0, None]
    o_dmas = [None, None]
    for ci in range(_NCHUNKS):
        b = ci % 2
        row0 = base_row + ci * _CHUNK
        if ci + 1 < _NCHUNKS:
            nb = (ci + 1) % 2
            x_dmas[nb] = pltpu.async_copy(
                x_hbm.at[pl.ds(row0 + _CHUNK, _CHUNK)], x_bufs[nb], x_sems[nb])
        x_dmas[b].wait()
        if o_dmas[b] is not None:
            o_dmas[b].wait()

        x_v = x_bufs[b]
        out_v = out_bufs[b]

        def row_body(i, carry):
            w_r, idx_r = carry
            for u in range(_U):
                r = i * _U + u
                vals = tuple(x_v[r, pl.ds(g * _L, _L)] * w_r[g]
                             for g in range(_G))
                rsplat = jnp.full((_L,), r, jnp.int32)
                for g in range(_G):
                    plsc.store_scatter(out_v, [rsplat, idx_r[g]], vals[g])
            return carry

        lax.fori_loop(0, _CHUNK // _U, row_body, (w_regs, idx_regs))

        o_dmas[b] = pltpu.async_copy(
            out_v, out_hbm.at[pl.ds(row0, _CHUNK)], o_sems[b])

    for d in o_dmas:
        if d is not None:
            d.wait()


def kernel(X_in, weights, input_node_order):
    mesh = plsc.VectorSubcoreMesh(
        core_axis_name="c", subcore_axis_name="s",
        num_cores=_NC, num_subcores=_NS,
    )
    f = pl.kernel(
        _sc_body,
        out_type=jax.ShapeDtypeStruct((_BATCH, _NOUT), jnp.float32),
        mesh=mesh,
        compiler_params=pltpu.CompilerParams(
            needs_layout_passes=False, use_tc_tiling_on_sc=True),
        scratch_types=[
            pltpu.VMEM((_CHUNK, _NIN), jnp.float32),
            pltpu.VMEM((_CHUNK, _NIN), jnp.float32),
            pltpu.VMEM((_CHUNK, _NOUT), jnp.float32),
            pltpu.VMEM((_CHUNK, _NOUT), jnp.float32),
            pltpu.VMEM((_NIN,), jnp.float32),
            pltpu.VMEM((_NIN,), jnp.int32),
            pltpu.SemaphoreType.DMA,
            pltpu.SemaphoreType.DMA,
            pltpu.SemaphoreType.DMA,
            pltpu.SemaphoreType.DMA,
        ],
    )
    return f(X_in, weights, input_node_order)
